# baseline (speedup 1.0000x reference)
import functools

import jax
import jax.numpy as jnp
from jax import lax
from jax.experimental import pallas as pl
from jax.experimental.pallas import tpu as pltpu
from jax.experimental.pallas import tpu_sc as plsc

B = 16384
D = 32
NC = 2
NS = 16
NW = NC * NS
BPW = B // NW
ICHUNK = 128
NCHUNK = BPW // ICHUNK
L = 16


def _wmf_body(uidx_hbm, iidx_hbm, utab_hbm, itab_hbm, out_hbm,
              uidx_v, iidx_v, urows_v, irows_v, out_v, sem_u, sem_i):
    wid = lax.axis_index("s") * NC + lax.axis_index("c")
    base = wid * BPW

    pltpu.sync_copy(uidx_hbm.at[wid], uidx_v)
    pltpu.sync_copy(iidx_hbm.at[wid], iidx_v)

    copies = []
    for j in range(NCHUNK):
        copies.append(pltpu.async_copy(
            utab_hbm.at[uidx_v.at[j]],
            urows_v.at[pl.ds(j * ICHUNK, ICHUNK)], sem_u))
        copies.append(pltpu.async_copy(
            itab_hbm.at[iidx_v.at[j]],
            irows_v.at[pl.ds(j * ICHUNK, ICHUNK)], sem_i))
    for c in copies:
        c.wait()

    lane = lax.iota(jnp.int32, L)

    def group(g, carry):
        rows = g * L + lane
        acc = jnp.zeros((L,), jnp.float32)
        for d in range(D):
            col = jnp.full((L,), d, jnp.int32)
            u = plsc.load_gather(urows_v, [rows, col])
            v = plsc.load_gather(irows_v, [rows, col])
            acc = acc + u * v
        out_v[pl.ds(g * L, L)] = acc
        return carry

    lax.fori_loop(0, BPW // L, group, 0)

    pltpu.sync_copy(out_v, out_hbm.at[pl.ds(base, BPW)])


@functools.partial(jax.jit, static_argnums=())
def kernel(user_input, item_input, user_table, item_table):
    uidx = user_input.astype(jnp.int32).reshape(NW, NCHUNK, ICHUNK)
    iidx = item_input.astype(jnp.int32).reshape(NW, NCHUNK, ICHUNK)
    mesh = plsc.VectorSubcoreMesh(core_axis_name="c", subcore_axis_name="s")
    f = functools.partial(
        pl.kernel,
        mesh=mesh,
        out_type=jax.ShapeDtypeStruct((B,), jnp.float32),
        scratch_types=[
            pltpu.VMEM((NCHUNK, ICHUNK), jnp.int32),
            pltpu.VMEM((NCHUNK, ICHUNK), jnp.int32),
            pltpu.VMEM((BPW, D), jnp.float32),
            pltpu.VMEM((BPW, D), jnp.float32),
            pltpu.VMEM((BPW,), jnp.float32),
            pltpu.SemaphoreType.DMA,
            pltpu.SemaphoreType.DMA,
        ],
        compiler_params=pltpu.CompilerParams(
            needs_layout_passes=False, use_tc_tiling_on_sc=False),
    )(_wmf_body)
    return f(uidx, iidx, user_table, item_table)


# speedup vs baseline: 3.2509x; 3.2509x over previous
import functools

import jax
import jax.numpy as jnp
from jax import lax
from jax.experimental import pallas as pl
from jax.experimental.pallas import tpu as pltpu
from jax.experimental.pallas import tpu_sc as plsc

B = 16384
D = 32
NC = 2
NS = 16
NW = NC * NS
BPW = B // NW
L = 16
KB = 16
NBATCH = BPW // KB


def _wmf_body(uidx_hbm, iidx_hbm, utab_hbm, itab_hbm, out_hbm,
              idx_v, blk_v, urow_v, out_v, sem0, sem1):
    wid = lax.axis_index("s") * NC + lax.axis_index("c")
    base = wid * BPW

    pltpu.sync_copy(uidx_hbm.at[wid], idx_v.at[0])
    pltpu.sync_copy(iidx_hbm.at[wid], idx_v.at[1])

    tabs = (utab_hbm, itab_hbm)
    sems = (sem0, sem1)

    def issue(t, a, b, slot):
        vec = idx_v[t, pl.ds(b * KB, KB)]
        for k in range(KB):
            start = pl.multiple_of((vec[k] // 128) * 128, 128)
            pltpu.async_copy(
                tabs[t].at[pl.ds(8 * a, 8), pl.ds(start, 128)],
                blk_v.at[slot, k], sems[slot])

    def drain(t, slot):
        for k in range(KB):
            pltpu.make_async_copy(
                tabs[t].at[pl.ds(0, 8), pl.ds(0, 128)],
                blk_v.at[slot, k], sems[slot]).wait()

    def extract(t, a, b, slot):
        vec = idx_v[t, pl.ds(b * KB, KB)]
        lane = lax.rem(vec, 128)
        k_v = lax.iota(jnp.int32, L)
        s_v = jnp.full((L,), slot, jnp.int32)
        if t == 1:
            acc = jnp.zeros((L,), jnp.float32)
        for dl in range(8):
            d_v = jnp.full((L,), dl, jnp.int32)
            val = plsc.load_gather(blk_v, [s_v, k_v, d_v, lane])
            if t == 0:
                urow_v[8 * a + dl, pl.ds(b * KB, KB)] = val
            else:
                acc = acc + urow_v[8 * a + dl, pl.ds(b * KB, KB)] * val
        if t == 1:
            if a == 0:
                out_v[pl.ds(b * KB, KB)] = acc
            else:
                out_v[pl.ds(b * KB, KB)] = out_v[pl.ds(b * KB, KB)] + acc

    for t in range(2):
        for a in range(4):
            issue(t, a, 0, 0)
            issue(t, a, 1, 1)

            def body(g, carry, t=t, a=a):
                drain(t, 0)
                extract(t, a, 2 * g, 0)

                @pl.when(g < NBATCH // 2 - 1)
                def _():
                    issue(t, a, 2 * g + 2, 0)

                drain(t, 1)
                extract(t, a, 2 * g + 1, 1)

                @pl.when(g < NBATCH // 2 - 1)
                def _():
                    issue(t, a, 2 * g + 3, 1)

                return carry

            lax.fori_loop(0, NBATCH // 2, body, 0)

    pltpu.sync_copy(out_v, out_hbm.at[pl.ds(base, BPW)])


@jax.jit
def kernel(user_input, item_input, user_table, item_table):
    uidx = user_input.astype(jnp.int32).reshape(NW, BPW)
    iidx = item_input.astype(jnp.int32).reshape(NW, BPW)
    mesh = plsc.VectorSubcoreMesh(core_axis_name="c", subcore_axis_name="s")
    f = functools.partial(
        pl.kernel,
        mesh=mesh,
        out_type=jax.ShapeDtypeStruct((B,), jnp.float32),
        scratch_types=[
            pltpu.VMEM((2, BPW), jnp.int32),
            pltpu.VMEM((2, KB, 8, 128), jnp.float32),
            pltpu.VMEM((D, BPW), jnp.float32),
            pltpu.VMEM((BPW,), jnp.float32),
            pltpu.SemaphoreType.DMA,
            pltpu.SemaphoreType.DMA,
        ],
        compiler_params=pltpu.CompilerParams(
            needs_layout_passes=False, use_tc_tiling_on_sc=True),
    )(_wmf_body)
    return f(uidx, iidx, user_table.T, item_table.T)


# speedup vs baseline: 3.5085x; 1.0793x over previous
import functools

import jax
import jax.numpy as jnp
from jax import lax
from jax.experimental import pallas as pl
from jax.experimental.pallas import tpu as pltpu
from jax.experimental.pallas import tpu_sc as plsc

B = 16384
D = 32
NC = 2
NS = 16
NW = NC * NS
BPW = B // NW
L = 16
KB = 8
NPAIR = BPW // (2 * KB)


def _wmf_body(uidx_hbm, iidx_hbm, utab_hbm, itab_hbm, out_hbm,
              idx_v, blk_v, urow_v, out_v, sem0, sem1):
    wid = lax.axis_index("s") * NC + lax.axis_index("c")
    base = wid * BPW

    pltpu.sync_copy(uidx_hbm.at[wid], idx_v.at[0])
    pltpu.sync_copy(iidx_hbm.at[wid], idx_v.at[1])

    tabs = (utab_hbm, itab_hbm)
    sems = (sem0, sem1)

    def issue_pair(t, g):
        vec = idx_v[t, pl.ds(g * 2 * KB, L)]
        for k in range(L):
            start = pl.multiple_of((vec[k] // 128) * 128, 128)
            pltpu.async_copy(
                tabs[t].at[:, pl.ds(start, 128)],
                blk_v.at[k // KB, k % KB], sems[k // KB])

    def drain(t, slot):
        for k in range(KB):
            pltpu.make_async_copy(
                tabs[t].at[:, pl.ds(0, 128)],
                blk_v.at[slot, k], sems[slot]).wait()

    def extract_pair(t, g):
        vec = idx_v[t, pl.ds(g * 2 * KB, L)]
        lane = lax.rem(vec, 128)
        j = lax.iota(jnp.int32, L)
        s_v = j // KB
        k_v = lax.rem(j, KB)
        sl = pl.ds(g * 2 * KB, L)
        if t == 1:
            acc = jnp.zeros((L,), jnp.float32)
        for d in range(D):
            d_v = jnp.full((L,), d, jnp.int32)
            val = plsc.load_gather(blk_v, [s_v, k_v, d_v, lane])
            if t == 0:
                urow_v[d, sl] = val
            else:
                acc = acc + urow_v[d, sl] * val
        if t == 1:
            out_v[sl] = acc

    for t in range(2):
        issue_pair(t, 0)

        def body(g, carry, t=t):
            drain(t, 0)
            drain(t, 1)
            extract_pair(t, g)

            @pl.when(g < NPAIR - 1)
            def _():
                issue_pair(t, g + 1)

            return carry

        lax.fori_loop(0, NPAIR, body, 0)

    pltpu.sync_copy(out_v, out_hbm.at[pl.ds(base, BPW)])


@jax.jit
def kernel(user_input, item_input, user_table, item_table):
    uidx = user_input.astype(jnp.int32).reshape(NW, BPW)
    iidx = item_input.astype(jnp.int32).reshape(NW, BPW)
    mesh = plsc.VectorSubcoreMesh(core_axis_name="c", subcore_axis_name="s")
    f = functools.partial(
        pl.kernel,
        mesh=mesh,
        out_type=jax.ShapeDtypeStruct((B,), jnp.float32),
        scratch_types=[
            pltpu.VMEM((2, BPW), jnp.int32),
            pltpu.VMEM((2, KB, D, 128), jnp.float32),
            pltpu.VMEM((D, BPW), jnp.float32),
            pltpu.VMEM((BPW,), jnp.float32),
            pltpu.SemaphoreType.DMA,
            pltpu.SemaphoreType.DMA,
        ],
        compiler_params=pltpu.CompilerParams(
            needs_layout_passes=False, use_tc_tiling_on_sc=True),
    )(_wmf_body)
    return f(uidx, iidx, user_table.T, item_table.T)
